# Initial kernel scaffold; baseline (speedup 1.0000x reference)
import jax
import jax.numpy as jnp
from jax.experimental import pallas as pl


def kernel(x, nodes_in, edge_index, edges_in, global_in, batch_index, params):
    raise NotImplementedError("write your pallas kernel here")


# speedup vs baseline: 3.7448x; 3.7448x over previous
import functools

import jax
import jax.numpy as jnp
from jax import lax
from jax.experimental import pallas as pl
from jax.experimental.pallas import tpu as pltpu
from jax.experimental.pallas import tpu_sc as plsc

N = 10000
E = 320000
D = 128
IN = 16

NC = 2
NS = 16
L = 16
NW = NC * NS
EPW = E // NW
CH = 80
NCHUNK = EPW // CH
NACC = 10240
NPT = NACC // NS

_f32 = jnp.float32


def _proj_body(nodes_ref, wv, bv, wp, bp, wa, ba, vo, po, ao):
    nb = nodes_ref[...]
    vo[...] = jnp.dot(nb, wv[...], preferred_element_type=_f32) + bv[...]
    po[...] = jnp.dot(nb, wp[...], preferred_element_type=_f32) + bp[...]
    ao[...] = jnp.dot(nb, wa[...], preferred_element_type=_f32) + ba[...]


def _project_nodes(nodes, wv, bv, wp, bp, wa, ba):
    blk = 1000
    grid = N // blk
    full = lambda shape: pl.BlockSpec(shape, lambda i: (0, 0))
    return pl.pallas_call(
        _proj_body,
        grid=(grid,),
        in_specs=[
            pl.BlockSpec((blk, D), lambda i: (i, 0)),
            full((D, D)), full((1, D)),
            full((D, D)), full((1, D)),
            full((D, D)), full((1, D)),
        ],
        out_specs=[pl.BlockSpec((blk, D), lambda i: (i, 0))] * 3,
        out_shape=[jax.ShapeDtypeStruct((N, D), _f32)] * 3,
    )(nodes, wv, bv, wp, bp, wa, ba)


def _gather_body(varphi, phi, alpha, x, row, col,
                 wsub_o, acol_o, b16_o,
                 ridx, cidx, vp, ph, al, xr, xc, b16b, sem):
    c = lax.axis_index("c")
    s = lax.axis_index("s")
    wid = s * NC + c

    def chunk(i, carry):
        base = wid * EPW + i * CH
        pltpu.sync_copy(row.at[pl.ds(base, CH)], ridx)
        pltpu.sync_copy(col.at[pl.ds(base, CH)], cidx)
        d1 = pltpu.async_copy(varphi.at[ridx], vp, sem)
        d2 = pltpu.async_copy(phi.at[cidx], ph, sem)
        d3 = pltpu.async_copy(alpha.at[cidx], al, sem)
        d4 = pltpu.async_copy(x.at[ridx], xr, sem)
        d5 = pltpu.async_copy(x.at[cidx], xc, sem)
        d1.wait(); d2.wait(); d3.wait(); d4.wait(); d5.wait()

        def esub(e, carry2):
            for j in range(D // L):
                sl = pl.ds(j * L, L)
                vp[e, sl] = vp[e, sl] - ph[e, sl]
            sl = pl.ds(0, L)
            b16b[e, sl] = xr[e, sl] - xc[e, sl]
            return carry2

        lax.fori_loop(0, CH, esub, 0)
        pltpu.sync_copy(vp, wsub_o.at[pl.ds(base, CH)])
        pltpu.sync_copy(al, acol_o.at[pl.ds(base, CH)])
        pltpu.sync_copy(b16b, b16_o.at[pl.ds(base, CH)])
        return carry

    lax.fori_loop(0, NCHUNK, chunk, 0)


def _gather_edges(varphi, phi, alpha, x, row, col):
    mesh = plsc.VectorSubcoreMesh(core_axis_name="c", subcore_axis_name="s")
    return pl.kernel(
        _gather_body,
        out_type=[
            jax.ShapeDtypeStruct((E, D), _f32),
            jax.ShapeDtypeStruct((E, D), _f32),
            jax.ShapeDtypeStruct((E, IN), _f32),
        ],
        mesh=mesh,
        scratch_types=[
            pltpu.VMEM((CH,), jnp.int32),
            pltpu.VMEM((CH,), jnp.int32),
            pltpu.VMEM((CH, D), _f32),
            pltpu.VMEM((CH, D), _f32),
            pltpu.VMEM((CH, D), _f32),
            pltpu.VMEM((CH, D), _f32),
            pltpu.VMEM((CH, D), _f32),
            pltpu.VMEM((CH, IN), _f32),
            pltpu.SemaphoreType.DMA,
        ],
    )(varphi, phi, alpha, x, row, col)


def _edge_body(b16_ref, wsub_ref, ein_ref, acol_ref,
               dW1, db1, dW2, db2, dW3, db3,
               gW1a, gW1b, gb1, gW2, gb2, gW3, gb3,
               eo_ref, ex_ref, exv_ref):
    dot = lambda a, b: jnp.dot(a, b, preferred_element_type=_f32)
    g = jnp.maximum(dot(b16_ref[...], dW1[...]) + db1[...], 0.0)
    g = jnp.maximum(dot(g, dW2[...]) + db2[...], 0.0)
    delta = dot(g, dW3[...]) + db3[...]
    h = wsub_ref[...] + ein_ref[...]
    u = jnp.maximum(dot(h, gW1a[...]) + dot(delta, gW1b[...]) + gb1[...], 0.0)
    u = jnp.maximum(dot(u, gW2[...]) + gb2[...], 0.0)
    eo = dot(u, gW3[...]) + gb3[...]
    ex = jnp.exp(eo)
    eo_ref[...] = eo
    ex_ref[...] = ex
    exv_ref[...] = ex * (acol_ref[...] + delta)


def _edge_mlps(b16, wsub, edges_in, acol, dws, dbs, gws, gbs):
    blk = 512
    grid = E // blk
    full = lambda shape: pl.BlockSpec(shape, lambda i: (0, 0))
    dW1, dW2, dW3 = dws
    db1, db2, db3 = [b.reshape(1, -1) for b in dbs]
    gW1, gW2, gW3 = gws
    gb1, gb2, gb3 = [b.reshape(1, -1) for b in gbs]
    gW1a = gW1[:D]
    gW1b = gW1[D:]
    H = dW2.shape[0]
    eblk = lambda w: pl.BlockSpec((blk, w), lambda i: (i, 0))
    return pl.pallas_call(
        _edge_body,
        grid=(grid,),
        in_specs=[
            eblk(IN), eblk(D), eblk(D), eblk(D),
            full((IN, H)), full((1, H)), full((H, H)), full((1, H)),
            full((H, D)), full((1, D)),
            full((D, H)), full((D, H)), full((1, H)),
            full((H, H)), full((1, H)), full((H, D)), full((1, D)),
        ],
        out_specs=[eblk(D)] * 3,
        out_shape=[jax.ShapeDtypeStruct((E, D), _f32)] * 3,
    )(b16, wsub, edges_in, acol,
      dW1, db1, dW2, db2, dW3, db3,
      gW1a, gW1b, gb1, gW2, gb2, gW3, gb3)


def _scatter_body(vals, row, zeros, out, ridx, vbuf, acc, sem):
    c = lax.axis_index("c")
    s = lax.axis_index("s")
    pltpu.sync_copy(zeros, acc.at[pl.ds(s * NPT, NPT)])
    plsc.subcore_barrier()
    epw = E // NC // NS

    def chunk(i, carry):
        base = (c * NS + s) * epw + i * CH
        pltpu.sync_copy(row.at[pl.ds(base, CH)], ridx)
        pltpu.sync_copy(vals.at[pl.ds(base, CH)], vbuf)
        pltpu.sync_copy(vbuf, acc.at[ridx], add=True)
        return carry

    lax.fori_loop(0, NCHUNK, chunk, 0)
    plsc.subcore_barrier()
    pltpu.sync_copy(acc.at[pl.ds(s * NPT, NPT)],
                    out.at[c, pl.ds(s * NPT, NPT)])


def _segment_sum(vals, row, zeros):
    mesh = plsc.VectorSubcoreMesh(core_axis_name="c", subcore_axis_name="s")
    return pl.kernel(
        _scatter_body,
        out_type=jax.ShapeDtypeStruct((NC, NACC, D), _f32),
        mesh=mesh,
        scratch_types=[
            pltpu.VMEM((CH,), jnp.int32),
            pltpu.VMEM((CH, D), _f32),
            pltpu.VMEM_SHARED((NACC, D), _f32),
            pltpu.SemaphoreType.DMA,
        ],
    )(vals, row, zeros)


def _node_body(dp_ref, np_ref, nodes_ref,
               bW1a, bW1b, bb1, bW2, bb2, bW3, bb3, out_ref):
    dot = lambda a, b: jnp.dot(a, b, preferred_element_type=_f32)
    den = dp_ref[0] + dp_ref[1]
    num = np_ref[0] + np_ref[1]
    agg = jnp.where(den > 0.0, num / den, 0.0)
    u = jnp.maximum(dot(agg, bW1a[...]) + dot(nodes_ref[...], bW1b[...])
                    + bb1[...], 0.0)
    u = jnp.maximum(dot(u, bW2[...]) + bb2[...], 0.0)
    out_ref[...] = dot(u, bW3[...]) + bb3[...]


def _node_mlp(den_parts, num_parts, nodes, bws, bbs):
    blk = 1000
    grid = N // blk
    full = lambda shape: pl.BlockSpec(shape, lambda i: (0, 0))
    bW1, bW2, bW3 = bws
    bb1, bb2, bb3 = [b.reshape(1, -1) for b in bbs]
    bW1a = bW1[:D]
    bW1b = bW1[D:]
    H = bW2.shape[0]
    pblk = pl.BlockSpec((NC, blk, D), lambda i: (0, i, 0))
    return pl.pallas_call(
        _node_body,
        grid=(grid,),
        in_specs=[
            pblk, pblk, pl.BlockSpec((blk, D), lambda i: (i, 0)),
            full((D, H)), full((D, H)), full((1, H)),
            full((H, H)), full((1, H)), full((H, D)), full((1, D)),
        ],
        out_specs=pl.BlockSpec((blk, D), lambda i: (i, 0)),
        out_shape=jax.ShapeDtypeStruct((N, D), _f32),
    )(den_parts, num_parts, nodes, bW1a, bW1b, bb1, bW2, bb2, bW3, bb3)


def kernel(x, nodes_in, edge_index, edges_in, global_in, batch_index, params):
    row = edge_index[0]
    col = edge_index[1]

    varphi, phi, alpha = _project_nodes(
        nodes_in,
        params['varphi_W'], params['varphi_b'].reshape(1, -1),
        params['phi_W'], params['phi_b'].reshape(1, -1),
        params['alpha_W'], params['alpha_b'].reshape(1, -1))

    xpad = jnp.pad(x, ((0, 0), (0, D - IN)))
    wsub, acol, b16 = _gather_edges(varphi, phi, alpha, xpad, row, col)

    edges_out, ex, exv = _edge_mlps(
        b16, wsub, edges_in, acol,
        params['delta_Ws'], params['delta_bs'],
        params['gamma_Ws'], params['gamma_bs'])

    zeros = jnp.zeros((NPT, D), _f32)
    den_parts = _segment_sum(ex, row, zeros)
    num_parts = _segment_sum(exv, row, zeros)

    nodes_out = _node_mlp(den_parts, num_parts, nodes_in,
                          params['beta_Ws'], params['beta_bs'])
    return nodes_out, edges_out


# speedup vs baseline: 5.0254x; 1.3420x over previous
import jax
import jax.numpy as jnp
from jax import lax
from jax.experimental import pallas as pl
from jax.experimental.pallas import tpu as pltpu
from jax.experimental.pallas import tpu_sc as plsc

N = 10000
E = 320000
D = 128
IN = 16

NC = 2
NS = 16
L = 16
NW = NC * NS
EPW = E // NW
CHB = 40
NCHB = EPW // CHB
CHD = 80
NCHD = EPW // CHD
NACC = 10240
NPT = NACC // NS

_f32 = jnp.float32


def _mesh():
    return plsc.VectorSubcoreMesh(core_axis_name="c", subcore_axis_name="s")


def _proj_body(nodes_ref, wv, bv, wp, bp, wa, ba, vo, pa_o):
    nb = nodes_ref[...]
    vo[...] = jnp.dot(nb, wv[...], preferred_element_type=_f32) + bv[...]
    pa_o[:, :D] = jnp.dot(nb, wp[...], preferred_element_type=_f32) + bp[...]
    pa_o[:, D:] = jnp.dot(nb, wa[...], preferred_element_type=_f32) + ba[...]


def _project_nodes(nodes, wv, bv, wp, bp, wa, ba):
    blk = 1000
    full = lambda shape: pl.BlockSpec(shape, lambda i: (0, 0))
    return pl.pallas_call(
        _proj_body,
        grid=(N // blk,),
        in_specs=[
            pl.BlockSpec((blk, D), lambda i: (i, 0)),
            full((D, D)), full((1, D)),
            full((D, D)), full((1, D)),
            full((D, D)), full((1, D)),
        ],
        out_specs=[pl.BlockSpec((blk, D), lambda i: (i, 0)),
                   pl.BlockSpec((blk, 2 * D), lambda i: (i, 0))],
        out_shape=[jax.ShapeDtypeStruct((N, D), _f32),
                   jax.ShapeDtypeStruct((N, 2 * D), _f32)],
    )(nodes, wv, bv, wp, bp, wa, ba)


NBUF_B = 3


def _gather_body(varphi, phial, x2, row2, col2, g_o, b16_o,
                 ridx, cidx, vps, tcs, xrs, xcs, b16s, gsems, wsems, lsem):
    c = lax.axis_index("c")
    s = lax.axis_index("s")
    wid = s * NC + c

    pltpu.async_copy(row2.at[wid], ridx, lsem)
    pltpu.async_copy(col2.at[wid], cidx, lsem)
    pltpu.make_async_copy(row2.at[wid], ridx, lsem).wait()
    pltpu.make_async_copy(col2.at[wid], cidx, lsem).wait()

    def issue(i, b):
        ri = ridx.at[pl.ds(i * CHB, CHB)]
        ci = cidx.at[pl.ds(i * CHB, CHB)]
        pltpu.async_copy(phial.at[ci], tcs[b], gsems[b])
        pltpu.async_copy(varphi.at[ri], vps[b], gsems[b])
        pltpu.async_copy(x2.at[ri], xrs[b], gsems[b])
        pltpu.async_copy(x2.at[ci], xcs[b], gsems[b])

    def wait_gathers(b):
        pltpu.make_async_copy(phial.at[pl.ds(0, CHB)], tcs[b],
                              gsems[b]).wait()
        pltpu.make_async_copy(varphi.at[pl.ds(0, CHB)], vps[b],
                              gsems[b]).wait()
        pltpu.make_async_copy(x2.at[pl.ds(0, CHB)], xrs[b], gsems[b]).wait()
        pltpu.make_async_copy(x2.at[pl.ds(0, CHB)], xcs[b], gsems[b]).wait()

    def compute(b):
        vp, tc, xr, xc, b16 = vps[b], tcs[b], xrs[b], xcs[b], b16s[b]

        def esub(e, carry):
            for j in range(D // L):
                sl = pl.ds(j * L, L)
                tc[e, sl] = vp[e, sl] - tc[e, sl]
            sl = pl.ds(0, L)
            b16[e, :] = xr[e, sl] - xc[e, sl]
            return carry

        lax.fori_loop(0, CHB, esub, 0)

    def issue_wb(i, b):
        base = wid * EPW + i * CHB
        pltpu.async_copy(tcs[b], g_o.at[pl.ds(base, CHB)], wsems[b])
        pltpu.async_copy(b16s[b], b16_o.at[pl.ds(base, CHB)], wsems[b])

    def drain_wb(b):
        pltpu.make_async_copy(tcs[b], g_o.at[pl.ds(0, CHB)],
                              wsems[b]).wait()
        pltpu.make_async_copy(b16s[b], b16_o.at[pl.ds(0, CHB)],
                              wsems[b]).wait()

    def step(i, b):
        bn = (b + 1) % NBUF_B

        @pl.when(i + 1 < NCHB)
        def _():
            @pl.when(i >= 2)
            def _():
                drain_wb(bn)
            issue(i + 1, bn)

        wait_gathers(b)
        compute(b)
        issue_wb(i, b)

    issue(0, 0)
    trips = (NCHB + NBUF_B - 1) // NBUF_B

    def trip(j, carry):
        for b in range(NBUF_B):
            i = j * NBUF_B + b

            @pl.when(i < NCHB)
            def _():
                step(i, b)
        return carry

    lax.fori_loop(0, trips, trip, 0)
    for k in range(NBUF_B):
        drain_wb((NCHB - 1 - k) % NBUF_B)


def _gather_edges(varphi, phial, x2, row2, col2):
    return pl.kernel(
        _gather_body,
        out_type=[
            jax.ShapeDtypeStruct((E, 2 * D), _f32),
            jax.ShapeDtypeStruct((E, IN), _f32),
        ],
        mesh=_mesh(),
        scratch_types=[
            pltpu.VMEM((EPW,), jnp.int32),
            pltpu.VMEM((EPW,), jnp.int32),
            [pltpu.VMEM((CHB, D), _f32)] * NBUF_B,
            [pltpu.VMEM((CHB, 2 * D), _f32)] * NBUF_B,
            [pltpu.VMEM((CHB, D), _f32)] * NBUF_B,
            [pltpu.VMEM((CHB, D), _f32)] * NBUF_B,
            [pltpu.VMEM((CHB, IN), _f32)] * NBUF_B,
            [pltpu.SemaphoreType.DMA] * NBUF_B,
            [pltpu.SemaphoreType.DMA] * NBUF_B,
            pltpu.SemaphoreType.DMA,
        ],
    )(varphi, phial, x2, row2, col2)


def _edge_body(b16_ref, g_ref, ein_ref,
               dW1, db1, dW2, db2, dW3, db3,
               gW1a, gW1b, gb1, gW2, gb2, gW3, gb3,
               eo_ref, ex_ref, exv_ref):
    dot = lambda a, b: jnp.dot(a, b, preferred_element_type=_f32)
    g = jnp.maximum(dot(b16_ref[...], dW1[...]) + db1[...], 0.0)
    g = jnp.maximum(dot(g, dW2[...]) + db2[...], 0.0)
    delta = dot(g, dW3[...]) + db3[...]
    h = g_ref[:, :D] + ein_ref[...]
    u = jnp.maximum(dot(h, gW1a[...]) + dot(delta, gW1b[...]) + gb1[...], 0.0)
    u = jnp.maximum(dot(u, gW2[...]) + gb2[...], 0.0)
    eo = dot(u, gW3[...]) + gb3[...]
    ex = jnp.exp(eo)
    eo_ref[...] = eo
    ex_ref[...] = ex
    exv_ref[...] = ex * (g_ref[:, D:] + delta)


def _edge_mlps(b16, g, edges_in, dws, dbs, gws, gbs):
    blk = 512
    full = lambda shape: pl.BlockSpec(shape, lambda i: (0, 0))
    dW1, dW2, dW3 = dws
    db1, db2, db3 = [b.reshape(1, -1) for b in dbs]
    gW1, gW2, gW3 = gws
    gb1, gb2, gb3 = [b.reshape(1, -1) for b in gbs]
    gW1a = gW1[:D]
    gW1b = gW1[D:]
    H = dW2.shape[0]
    eblk = lambda w: pl.BlockSpec((blk, w), lambda i: (i, 0))
    return pl.pallas_call(
        _edge_body,
        grid=(E // blk,),
        in_specs=[
            eblk(IN), eblk(2 * D), eblk(D),
            full((IN, H)), full((1, H)), full((H, H)), full((1, H)),
            full((H, D)), full((1, D)),
            full((D, H)), full((D, H)), full((1, H)),
            full((H, H)), full((1, H)), full((H, D)), full((1, D)),
        ],
        out_specs=[eblk(D)] * 3,
        out_shape=[jax.ShapeDtypeStruct((E, D), _f32)] * 3,
    )(b16, g, edges_in,
      dW1, db1, dW2, db2, dW3, db3,
      gW1a, gW1b, gb1, gW2, gb2, gW3, gb3)


NBUF_D = 3


def _scatter_pass(vals, rowf, out, zeros, idxbufs, vbufs, rsems, ssems,
                  acc, c, s):
    pltpu.sync_copy(zeros, acc.at[pl.ds(s * NPT, NPT)])
    plsc.subcore_barrier()
    wid = s * NC + c
    base0 = wid * EPW

    def issue_read(i, b):
        pltpu.async_copy(rowf.at[pl.ds(base0 + i * CHD, CHD)], idxbufs[b],
                         rsems[b])
        pltpu.async_copy(vals.at[pl.ds(base0 + i * CHD, CHD)], vbufs[b],
                         rsems[b])

    def wait_read(i, b):
        pltpu.make_async_copy(rowf.at[pl.ds(base0 + i * CHD, CHD)],
                              idxbufs[b], rsems[b]).wait()
        pltpu.make_async_copy(vals.at[pl.ds(base0 + i * CHD, CHD)], vbufs[b],
                              rsems[b]).wait()

    def issue_scatter(i, b):
        pltpu.async_copy(vbufs[b], acc.at[idxbufs[b]], ssems[b], add=True)

    def drain_scatter(b):
        pltpu.make_async_copy(vals.at[pl.ds(0, CHD)], vbufs[b],
                              ssems[b]).wait()

    def step(i, b):
        bn = (b + 1) % NBUF_D

        @pl.when(i + 1 < NCHD)
        def _():
            @pl.when(i >= NBUF_D - 1)
            def _():
                drain_scatter(bn)
            issue_read(i + 1, bn)

        wait_read(i, b)
        issue_scatter(i, b)

    issue_read(0, 0)
    trips = (NCHD + NBUF_D - 1) // NBUF_D

    def trip(j, carry):
        for b in range(NBUF_D):
            i = j * NBUF_D + b

            @pl.when(i < NCHD)
            def _():
                step(i, b)
        return carry

    lax.fori_loop(0, trips, trip, 0)
    for k in range(NBUF_D):
        drain_scatter((NCHD - 1 - k) % NBUF_D)
    plsc.subcore_barrier()
    pltpu.sync_copy(acc.at[pl.ds(s * NPT, NPT)],
                    out.at[c, pl.ds(s * NPT, NPT)])
    plsc.subcore_barrier()


def _seg_body(ex, exv, rowf, zeros, den_o, num_o,
              idxbufs, vbufs, acc, rsems, ssems):
    c = lax.axis_index("c")
    s = lax.axis_index("s")
    _scatter_pass(ex, rowf, den_o, zeros, idxbufs, vbufs, rsems, ssems,
                  acc, c, s)
    _scatter_pass(exv, rowf, num_o, zeros, idxbufs, vbufs, rsems, ssems,
                  acc, c, s)


def _segment_sums(ex, exv, rowf, zeros):
    return pl.kernel(
        _seg_body,
        out_type=[jax.ShapeDtypeStruct((NC, NACC, D), _f32)] * 2,
        mesh=_mesh(),
        scratch_types=[
            [pltpu.VMEM((CHD,), jnp.int32)] * NBUF_D,
            [pltpu.VMEM((CHD, D), _f32)] * NBUF_D,
            pltpu.VMEM_SHARED((NACC, D), _f32),
            [pltpu.SemaphoreType.DMA] * NBUF_D,
            [pltpu.SemaphoreType.DMA] * NBUF_D,
        ],
    )(ex, exv, rowf, zeros)


def _node_body(dp_ref, np_ref, nodes_ref,
               bW1a, bW1b, bb1, bW2, bb2, bW3, bb3, out_ref):
    dot = lambda a, b: jnp.dot(a, b, preferred_element_type=_f32)
    den = dp_ref[0] + dp_ref[1]
    num = np_ref[0] + np_ref[1]
    agg = jnp.where(den > 0.0, num / den, 0.0)
    u = jnp.maximum(dot(agg, bW1a[...]) + dot(nodes_ref[...], bW1b[...])
                    + bb1[...], 0.0)
    u = jnp.maximum(dot(u, bW2[...]) + bb2[...], 0.0)
    out_ref[...] = dot(u, bW3[...]) + bb3[...]


def _node_mlp(den_parts, num_parts, nodes, bws, bbs):
    blk = 1000
    full = lambda shape: pl.BlockSpec(shape, lambda i: (0, 0))
    bW1, bW2, bW3 = bws
    bb1, bb2, bb3 = [b.reshape(1, -1) for b in bbs]
    bW1a = bW1[:D]
    bW1b = bW1[D:]
    H = bW2.shape[0]
    pblk = pl.BlockSpec((NC, blk, D), lambda i: (0, i, 0))
    return pl.pallas_call(
        _node_body,
        grid=(N // blk,),
        in_specs=[
            pblk, pblk, pl.BlockSpec((blk, D), lambda i: (i, 0)),
            full((D, H)), full((D, H)), full((1, H)),
            full((H, H)), full((1, H)), full((H, D)), full((1, D)),
        ],
        out_specs=pl.BlockSpec((blk, D), lambda i: (i, 0)),
        out_shape=jax.ShapeDtypeStruct((N, D), _f32),
    )(den_parts, num_parts, nodes, bW1a, bW1b, bb1, bW2, bb2, bW3, bb3)


def kernel(x, nodes_in, edge_index, edges_in, global_in, batch_index, params):
    row2 = edge_index[0].reshape(NW, EPW)
    col2 = edge_index[1].reshape(NW, EPW)
    x2 = jnp.pad(x, ((0, NACC - N), (0, D - IN)))

    varphi, phial = _project_nodes(
        nodes_in,
        params['varphi_W'], params['varphi_b'].reshape(1, -1),
        params['phi_W'], params['phi_b'].reshape(1, -1),
        params['alpha_W'], params['alpha_b'].reshape(1, -1))

    g, b16 = _gather_edges(varphi, phial, x2, row2, col2)

    edges_out, ex, exv = _edge_mlps(
        b16, g, edges_in,
        params['delta_Ws'], params['delta_bs'],
        params['gamma_Ws'], params['gamma_bs'])

    zeros = jnp.zeros((NPT, D), _f32)
    den_parts, num_parts = _segment_sums(ex, exv, edge_index[0], zeros)

    nodes_out = _node_mlp(den_parts, num_parts, nodes_in,
                          params['beta_Ws'], params['beta_bs'])
    return nodes_out, edges_out
